# speedup vs baseline: 5.1989x; 5.1989x over previous
import functools
import math

import jax
import jax.numpy as jnp
from jax import lax
from jax.experimental import pallas as pl
from jax.experimental.pallas import tpu as pltpu
from jax.experimental.pallas import tpu_sc as plsc

N = 10000
E = 320000
D = 128
BN_EPS = 1e-5

NC = 2
NS = 16
NW = NC * NS
CHUNK = 128

EDGES_PER_W = -(-E // (NW * CHUNK)) * CHUNK
E_PAD = EDGES_PER_W * NW
N_CHUNKS = EDGES_PER_W // CHUNK
ROWS_PER_TILE = 640
N_PAD = ROWS_PER_TILE * NS


def _seg_sum_body(with_counts, x_hbm, src_hbm, dst_hbm, z2d_hbm, z1d_hbm,
                  acc_out, cnt_out, src_v, dst_v, rows_v, ones_v, acc_sh,
                  cnt_sh, gsem):
    c = lax.axis_index("c")
    s = lax.axis_index("s")
    w = c * NS + s

    for t in range(ROWS_PER_TILE // CHUNK):
        pltpu.sync_copy(z2d_hbm, acc_sh.at[pl.ds(s * ROWS_PER_TILE + t * CHUNK, CHUNK)])
    if with_counts:
        pltpu.sync_copy(z1d_hbm, cnt_sh.at[pl.ds(s * ROWS_PER_TILE, ROWS_PER_TILE)])
        for j in range(8):
            ones_v[pl.ds(16 * j, 16)] = jnp.ones((16,), jnp.float32)
    pltpu.sync_copy(src_hbm.at[w], src_v)
    pltpu.sync_copy(dst_hbm.at[w], dst_v)
    plsc.subcore_barrier()

    def step(j, carry):
        pltpu.async_copy(x_hbm.at[src_v.at[j]], rows_v, gsem).wait()
        pltpu.sync_copy(rows_v, acc_sh.at[dst_v.at[j]], add=True)
        if with_counts:
            pltpu.sync_copy(ones_v, cnt_sh.at[dst_v.at[j]], add=True)
        return carry

    lax.fori_loop(0, N_CHUNKS, step, 0)
    plsc.subcore_barrier()

    lo = s * ROWS_PER_TILE
    pltpu.sync_copy(acc_sh.at[pl.ds(lo, ROWS_PER_TILE)],
                    acc_out.at[c, pl.ds(lo, ROWS_PER_TILE)])
    if with_counts:
        pltpu.sync_copy(cnt_sh.at[pl.ds(lo, ROWS_PER_TILE)],
                        cnt_out.at[c, pl.ds(lo, ROWS_PER_TILE)])


def _make_seg_sum(with_counts):
    mesh = plsc.VectorSubcoreMesh(core_axis_name="c", subcore_axis_name="s")
    return pl.kernel(
        functools.partial(_seg_sum_body, with_counts),
        out_type=(
            jax.ShapeDtypeStruct((NC, N_PAD, D), jnp.float32),
            jax.ShapeDtypeStruct((NC, N_PAD), jnp.float32),
        ),
        mesh=mesh,
        scratch_types=(
            pltpu.VMEM((N_CHUNKS, CHUNK), jnp.int32),
            pltpu.VMEM((N_CHUNKS, CHUNK), jnp.int32),
            pltpu.VMEM((CHUNK, D), jnp.float32),
            pltpu.VMEM((CHUNK,), jnp.float32),
            pltpu.VMEM_SHARED((N_PAD, D), jnp.float32),
            pltpu.VMEM_SHARED((N_PAD,), jnp.float32),
            pltpu.SemaphoreType.DMA,
        ),
    )


_seg_sum_cnt = _make_seg_sum(True)
_seg_sum = _make_seg_sum(False)

BN_ROWS = 1000


def _dense1_body(acc_ref, cnt_ref, x_ref, wl_ref, bl_ref, wr_ref, g_ref,
                 be_ref, o_ref):
    a = acc_ref[0] + acc_ref[1]
    cnt = cnt_ref[0] + cnt_ref[1]
    aggr = a / jnp.maximum(cnt, 1.0)
    h = (jnp.dot(aggr, wl_ref[...], preferred_element_type=jnp.float32)
         + bl_ref[...]
         + jnp.dot(x_ref[...], wr_ref[...], preferred_element_type=jnp.float32))
    h = h * (g_ref[...] / math.sqrt(1.0 + BN_EPS)) + be_ref[...]
    o_ref[...] = jnp.maximum(h, 0.0)


def _dense2_body(acc_ref, cnt_ref, x_ref, wl_ref, bl_ref, wr_ref, o_ref):
    a = acc_ref[0] + acc_ref[1]
    cnt = cnt_ref[0] + cnt_ref[1]
    aggr = a / jnp.maximum(cnt, 1.0)
    h = (jnp.dot(aggr, wl_ref[...], preferred_element_type=jnp.float32)
         + bl_ref[...]
         + jnp.dot(x_ref[...], wr_ref[...], preferred_element_type=jnp.float32))
    norm = jnp.sqrt(jnp.sum(h * h, axis=-1, keepdims=True))
    o_ref[...] = h / jnp.maximum(norm, 1e-12)


def _dense_call(body, n_extra):
    grid = N // BN_ROWS
    w_spec = pl.BlockSpec((D, D), lambda i: (0, 0))
    v_spec = pl.BlockSpec((1, D), lambda i: (0, 0))
    extra = [w_spec, v_spec, w_spec] + [v_spec] * n_extra
    return pl.pallas_call(
        body,
        grid=(grid,),
        in_specs=[
            pl.BlockSpec((NC, BN_ROWS, D), lambda i: (0, i, 0)),
            pl.BlockSpec((NC, BN_ROWS, 1), lambda i: (0, i, 0)),
            pl.BlockSpec((BN_ROWS, D), lambda i: (i, 0)),
        ] + extra,
        out_specs=pl.BlockSpec((BN_ROWS, D), lambda i: (i, 0)),
        out_shape=jax.ShapeDtypeStruct((N, D), jnp.float32),
    )


def kernel(x, edge_index, W1l, b1l, W1r, g1, be1, W2l, b2l, W2r):
    src = edge_index[0].astype(jnp.int32)
    dst = edge_index[1].astype(jnp.int32)
    pad = E_PAD - E
    src = jnp.concatenate([src, jnp.zeros((pad,), jnp.int32)])
    dst = jnp.concatenate([dst, jnp.full((pad,), N, jnp.int32)])
    src3 = src.reshape(NW, N_CHUNKS, CHUNK)
    dst3 = dst.reshape(NW, N_CHUNKS, CHUNK)
    z2d = jnp.zeros((CHUNK, D), jnp.float32)
    z1d = jnp.zeros((ROWS_PER_TILE,), jnp.float32)

    acc1, cnt = _seg_sum_cnt(x, src3, dst3, z2d, z1d)
    acc1 = acc1[:, :N]
    cntN = cnt[:, :N].reshape(NC, N, 1)

    h = _dense_call(_dense1_body, 2)(
        acc1, cntN, x, W1l, b1l.reshape(1, D), W1r, g1.reshape(1, D),
        be1.reshape(1, D))

    acc2, _ = _seg_sum(h, src3, dst3, z2d, z1d)
    acc2 = acc2[:, :N]

    out = _dense_call(_dense2_body, 0)(
        acc2, cntN, h, W2l, b2l.reshape(1, D), W2r)
    return out
